# Initial kernel scaffold; baseline (speedup 1.0000x reference)
import jax
import jax.numpy as jnp
from jax.experimental import pallas as pl


def kernel(xyz, points, W1, b1, W2, b2, W3, b3):
    raise NotImplementedError("write your pallas kernel here")


# speedup vs baseline: 12.2613x; 12.2613x over previous
import functools

import jax
import jax.numpy as jnp
from jax import lax
from jax.experimental import pallas as pl
from jax.experimental.pallas import tpu as pltpu
from jax.experimental.pallas import tpu_sc as plsc

B = 16
N = 4096
S = 1024
K = 32
R2 = 0.2 * 0.2
C1 = 32


def _emfma(y, acc):
    c = y * 4097.0
    yh = c - (c - y)
    yl = y - yh
    p = y * y
    e = ((yh * yh - p) + 2.0 * (yh * yl)) + yl * yl
    s1 = acc + p
    bb = s1 - acc
    e1 = (acc - (s1 - bb)) + (p - bb)
    return s1 + (e1 + e)


def _fps_body(xt_ref, out_ref, dist_ref):
    x = xt_ref[0]
    y = xt_ref[1]
    z = xt_ref[2]
    iota = lax.broadcasted_iota(jnp.int32, (B, N), 1)
    lane_s = lax.broadcasted_iota(jnp.int32, (B, S), 1)
    dist_ref[:] = jnp.full((B, N), 1e10, dtype=jnp.float32)

    def step(i, far):
        onehot = iota == far
        cx = jnp.sum(jnp.where(onehot, x, 0.0), axis=1, keepdims=True)
        cy = jnp.sum(jnp.where(onehot, y, 0.0), axis=1, keepdims=True)
        cz = jnp.sum(jnp.where(onehot, z, 0.0), axis=1, keepdims=True)
        dx = x - cx
        dy = y - cy
        dz = z - cz
        d = _emfma(dz, _emfma(dy, dx * dx))
        dist = jnp.minimum(dist_ref[:], d)
        dist_ref[:] = dist
        sel = lane_s == i
        cstack = jnp.stack([cx, cy, cz])
        out_ref[:] = jnp.where(sel[None], cstack, out_ref[:])
        m = jnp.max(dist, axis=1, keepdims=True)
        far = jnp.min(jnp.where(dist == m, iota, N), axis=1, keepdims=True)
        return far

    lax.fori_loop(0, S, step, jnp.zeros((B, 1), jnp.int32))


def _fps(xt):
    return pl.pallas_call(
        _fps_body,
        out_shape=jax.ShapeDtypeStruct((3, B, S), jnp.float32),
        scratch_shapes=[pltpu.VMEM((B, N), jnp.float32)],
    )(xt)


def _gtable_body(xyz_ref, pts_ref, w1a_ref, w1b_ref, b1_ref, g_ref):
    xc = xyz_ref[0, :, pl.ds(0, 1)]
    yc = xyz_ref[0, :, pl.ds(1, 1)]
    zc = xyz_ref[0, :, pl.ds(2, 1)]
    gx = (xc * w1a_ref[pl.ds(0, 1), :]
          + yc * w1a_ref[pl.ds(1, 1), :]
          + zc * w1a_ref[pl.ds(2, 1), :])
    gp = jnp.dot(pts_ref[0], w1b_ref[:], preferred_element_type=jnp.float32)
    gval = gx + gp + b1_ref[:]
    g_ref[0] = jnp.concatenate(
        [gval, jnp.zeros((N, 128 - C1), jnp.float32)], axis=1)


def _gtable(xyz, points, w1a, w1b, b1r):
    return pl.pallas_call(
        _gtable_body,
        grid=(B,),
        in_specs=[
            pl.BlockSpec((1, N, 3), lambda b: (b, 0, 0)),
            pl.BlockSpec((1, N, 32), lambda b: (b, 0, 0)),
            pl.BlockSpec((3, C1), lambda b: (0, 0)),
            pl.BlockSpec((32, C1), lambda b: (0, 0)),
            pl.BlockSpec((1, C1), lambda b: (0, 0)),
        ],
        out_specs=pl.BlockSpec((1, N, 128), lambda b: (b, 0, 0)),
        out_shape=jax.ShapeDtypeStruct((B, N, 128), jnp.float32),
    )(xyz, points, w1a, w1b, b1r)


def _sc_gather(xf, yf, zf, nxf, nyf, nzf, gflat):
    info = plsc.get_sparse_core_info()
    nw = info.num_cores * info.num_subcores
    spw = (B * S) // nw
    bpw = S // spw
    mesh = plsc.VectorSubcoreMesh(core_axis_name="c", subcore_axis_name="s")

    @functools.partial(
        pl.kernel,
        mesh=mesh,
        out_type=jax.ShapeDtypeStruct((B * S * K, 128), jnp.float32),
        compiler_params=pltpu.CompilerParams(needs_layout_passes=False),
        scratch_types=[
            pltpu.VMEM((N,), jnp.float32),
            pltpu.VMEM((N,), jnp.float32),
            pltpu.VMEM((N,), jnp.float32),
            pltpu.VMEM((N,), jnp.float32),
            pltpu.VMEM((spw + 16,), jnp.float32),
            pltpu.VMEM((spw + 16,), jnp.float32),
            pltpu.VMEM((spw + 16,), jnp.float32),
            pltpu.VMEM((spw + 16,), jnp.float32),
            pltpu.VMEM((N + 64,), jnp.int32),
            pltpu.VMEM((K,), jnp.int32),
            pltpu.VMEM((K, 128), jnp.float32),
            pltpu.SemaphoreType.DMA,
        ],
    )
    def body(x_h, y_h, z_h, nx_h, ny_h, nz_h, g_h, out_h,
             xv, yv, zv, np2v, nxv, nyv, nzv, nc2v, buf, idx32, dest, sem):
        wid = lax.axis_index("s") * info.num_cores + lax.axis_index("c")
        b = wid // bpw
        pltpu.sync_copy(x_h.at[pl.ds(b * N, N)], xv)
        pltpu.sync_copy(y_h.at[pl.ds(b * N, N)], yv)
        pltpu.sync_copy(z_h.at[pl.ds(b * N, N)], zv)
        pltpu.sync_copy(nx_h.at[pl.ds(wid * spw, spw)], nxv.at[pl.ds(0, spw)])
        pltpu.sync_copy(ny_h.at[pl.ds(wid * spw, spw)], nyv.at[pl.ds(0, spw)])
        pltpu.sync_copy(nz_h.at[pl.ds(wid * spw, spw)], nzv.at[pl.ds(0, spw)])
        iota16 = lax.iota(jnp.int32, 16)
        grow = b * N

        def rbf16(v):
            u = lax.bitcast_convert_type(v, jnp.uint32)
            r = (u + jnp.uint32(0x7FFF) + ((u >> jnp.uint32(16))
                                           & jnp.uint32(1)))
            return lax.bitcast_convert_type(r & jnp.uint32(0xFFFF0000),
                                            jnp.float32)

        def prep_pts(c, carry):
            sl = pl.ds(c * 16, 16)
            x = xv[sl]
            y = yv[sl]
            z = zv[sl]
            np2v[sl] = x * x + y * y + z * z
            xv[sl] = rbf16(x)
            yv[sl] = rbf16(y)
            zv[sl] = rbf16(z)
            return carry

        lax.fori_loop(0, N // 16, prep_pts, jnp.int32(0))

        def prep_ctr(c, carry):
            sl = pl.ds(c * 16, 16)
            x = nxv[sl]
            y = nyv[sl]
            z = nzv[sl]
            nc2v[sl] = x * x + y * y + z * z
            nxv[sl] = rbf16(x)
            nyv[sl] = rbf16(y)
            nzv[sl] = rbf16(z)
            return carry

        lax.fori_loop(0, spw // 16, prep_ctr, jnp.int32(0))

        def per_center(s, carry):
            cx = nxv[pl.ds(s, 16)][0]
            cy = nyv[pl.ds(s, 16)][0]
            cz = nzv[pl.ds(s, 16)][0]
            nc2 = nc2v[pl.ds(s, 16)][0]
            buf[pl.ds(0, 16)] = jnp.zeros((16,), jnp.int32)

            def scan_grp(grp, cnt):
                for u in range(4):
                    off = grp * 64 + u * 16
                    sl = pl.ds(off, 16)
                    dot = xv[sl] * cx + yv[sl] * cy + zv[sl] * cz
                    sqr = (nc2 + np2v[sl]) - 2.0 * dot
                    m = sqr <= R2
                    mi = m.astype(jnp.int32)
                    rank = jnp.cumsum(mi)
                    plsc.store_scatter(buf, [cnt + rank - 1], off + iota16,
                                       mask=m)
                    cnt = cnt + jnp.sum(mi)
                return cnt

            cnt = lax.fori_loop(0, N // 64, scan_grp, jnp.int32(0))
            pv = jnp.full((16,), buf[pl.ds(0, 16)][0], jnp.int32)
            plsc.store_scatter(buf, [cnt + iota16], pv)
            plsc.store_scatter(buf, [cnt + 16 + iota16], pv)
            idx32[pl.ds(0, 16)] = buf[pl.ds(0, 16)] + grow
            idx32[pl.ds(16, 16)] = buf[pl.ds(16, 16)] + grow
            pltpu.async_copy(g_h.at[idx32], dest, sem).wait()
            pltpu.sync_copy(dest, out_h.at[pl.ds((wid * spw + s) * K, K)])
            return carry

        lax.fori_loop(0, spw, per_center, jnp.int32(0))

    return body(xf, yf, zf, nxf, nyf, nzf, gflat)


_ST = 256


def _mlp_body(g_ref, nx_ref, ny_ref, nz_ref, w1a_ref, w2_ref, b2_ref,
              w3_ref, b3_ref, out_ref):
    nxc = nx_ref[0, 0][:, None]
    nyc = ny_ref[0, 0][:, None]
    nzc = nz_ref[0, 0][:, None]
    c = (nxc * w1a_ref[pl.ds(0, 1), :]
         + nyc * w1a_ref[pl.ds(1, 1), :]
         + nzc * w1a_ref[pl.ds(2, 1), :])
    h1 = jax.nn.relu(g_ref[0][:, :, 0:C1] - c[:, None, :])
    h1f = h1.reshape(_ST * K, C1)
    h2 = jax.nn.relu(jnp.dot(h1f, w2_ref[:], preferred_element_type=jnp.float32)
                     + b2_ref[:])
    h3 = jax.nn.relu(jnp.dot(h2, w3_ref[:], preferred_element_type=jnp.float32)
                     + b3_ref[:])
    out_ref[0] = jnp.max(h3.reshape(_ST, K, 64), axis=1)


def _mlp(g4, nx, ny, nz, w1a, w2, b2r, w3, b3r):
    nt = S // _ST
    nx = nx.reshape(B * nt, 1, _ST)
    ny = ny.reshape(B * nt, 1, _ST)
    nz = nz.reshape(B * nt, 1, _ST)
    return pl.pallas_call(
        _mlp_body,
        grid=(B, nt),
        in_specs=[
            pl.BlockSpec((1, _ST, K, 128), lambda b, t: (b, t, 0, 0)),
            pl.BlockSpec((1, 1, _ST), lambda b, t: (b * nt + t, 0, 0)),
            pl.BlockSpec((1, 1, _ST), lambda b, t: (b * nt + t, 0, 0)),
            pl.BlockSpec((1, 1, _ST), lambda b, t: (b * nt + t, 0, 0)),
            pl.BlockSpec((3, C1), lambda b, t: (0, 0)),
            pl.BlockSpec((C1, 32), lambda b, t: (0, 0)),
            pl.BlockSpec((1, 32), lambda b, t: (0, 0)),
            pl.BlockSpec((32, 64), lambda b, t: (0, 0)),
            pl.BlockSpec((1, 64), lambda b, t: (0, 0)),
        ],
        out_specs=pl.BlockSpec((1, _ST, 64), lambda b, t: (b, t, 0)),
        out_shape=jax.ShapeDtypeStruct((B, S, 64), jnp.float32),
    )(g4, nx, ny, nz, w1a, w2, b2r, w3, b3r)


def kernel(xyz, points, W1, b1, W2, b2, W3, b3):
    xt = jnp.transpose(xyz, (2, 0, 1))
    newt = _fps(xt)
    new_xyz = jnp.transpose(newt, (1, 2, 0))
    w1a = W1[:3]
    w1b = W1[3:]
    g = _gtable(xyz, points, w1a, w1b, b1.reshape(1, C1))
    gathered = _sc_gather(
        xt[0].reshape(B * N), xt[1].reshape(B * N), xt[2].reshape(B * N),
        newt[0].reshape(B * S), newt[1].reshape(B * S), newt[2].reshape(B * S),
        g.reshape(B * N, 128),
    )
    g4 = gathered.reshape(B, S, K, 128)
    out = _mlp(g4, newt[0], newt[1], newt[2], w1a, W2,
               b2.reshape(1, 32), W3, b3.reshape(1, 64))
    return (new_xyz, out)


# speedup vs baseline: 17.9082x; 1.4605x over previous
import functools

import jax
import jax.numpy as jnp
from jax import lax
from jax.experimental import pallas as pl
from jax.experimental.pallas import tpu as pltpu
from jax.experimental.pallas import tpu_sc as plsc

B = 16
N = 4096
S = 1024
K = 32
R2 = 0.2 * 0.2
C1 = 32


def _emfma(y, acc):
    c = y * 4097.0
    yh = c - (c - y)
    yl = y - yh
    p = y * y
    e = ((yh * yh - p) + 2.0 * (yh * yl)) + yl * yl
    s1 = acc + p
    bb = s1 - acc
    e1 = (acc - (s1 - bb)) + (p - bb)
    return s1 + (e1 + e)


def _fps_body(xt_ref, out_ref, dist_ref):
    x = xt_ref[0]
    y = xt_ref[1]
    z = xt_ref[2]
    iota = lax.broadcasted_iota(jnp.int32, (B, N), 1)
    lane_s = lax.broadcasted_iota(jnp.int32, (B, S), 1)
    dist_ref[:] = jnp.full((B, N), 1e10, dtype=jnp.float32)

    def step(i, far):
        onehot = iota == far
        cx = jnp.sum(jnp.where(onehot, x, 0.0), axis=1, keepdims=True)
        cy = jnp.sum(jnp.where(onehot, y, 0.0), axis=1, keepdims=True)
        cz = jnp.sum(jnp.where(onehot, z, 0.0), axis=1, keepdims=True)
        dx = x - cx
        dy = y - cy
        dz = z - cz
        d = _emfma(dz, _emfma(dy, dx * dx))
        dist = jnp.minimum(dist_ref[:], d)
        dist_ref[:] = dist
        sel = lane_s == i
        cstack = jnp.stack([cx, cy, cz])
        out_ref[:] = jnp.where(sel[None], cstack, out_ref[:])
        m = jnp.max(dist, axis=1, keepdims=True)
        far = jnp.min(jnp.where(dist == m, iota, N), axis=1, keepdims=True)
        return far

    lax.fori_loop(0, S, step, jnp.zeros((B, 1), jnp.int32))


def _fps(xt):
    return pl.pallas_call(
        _fps_body,
        out_shape=jax.ShapeDtypeStruct((3, B, S), jnp.float32),
        scratch_shapes=[pltpu.VMEM((B, N), jnp.float32)],
    )(xt)


def _gtable_body(xyz_ref, pts_ref, w1a_ref, w1b_ref, b1_ref, g_ref):
    xc = xyz_ref[0, :, pl.ds(0, 1)]
    yc = xyz_ref[0, :, pl.ds(1, 1)]
    zc = xyz_ref[0, :, pl.ds(2, 1)]
    gx = (xc * w1a_ref[pl.ds(0, 1), :]
          + yc * w1a_ref[pl.ds(1, 1), :]
          + zc * w1a_ref[pl.ds(2, 1), :])
    gp = jnp.dot(pts_ref[0], w1b_ref[:], preferred_element_type=jnp.float32)
    gval = gx + gp + b1_ref[:]
    g_ref[0] = jnp.concatenate(
        [gval, jnp.zeros((N, 128 - C1), jnp.float32)], axis=1)


def _gtable(xyz, points, w1a, w1b, b1r):
    return pl.pallas_call(
        _gtable_body,
        grid=(B,),
        in_specs=[
            pl.BlockSpec((1, N, 3), lambda b: (b, 0, 0)),
            pl.BlockSpec((1, N, 32), lambda b: (b, 0, 0)),
            pl.BlockSpec((3, C1), lambda b: (0, 0)),
            pl.BlockSpec((32, C1), lambda b: (0, 0)),
            pl.BlockSpec((1, C1), lambda b: (0, 0)),
        ],
        out_specs=pl.BlockSpec((1, N, 128), lambda b: (b, 0, 0)),
        out_shape=jax.ShapeDtypeStruct((B, N, 128), jnp.float32),
    )(xyz, points, w1a, w1b, b1r)


def _sc_gather(xf, yf, zf, nxf, nyf, nzf, gflat):
    info = plsc.get_sparse_core_info()
    nw = info.num_cores * info.num_subcores
    spw = (B * S) // nw
    bpw = S // spw
    mesh = plsc.VectorSubcoreMesh(core_axis_name="c", subcore_axis_name="s")

    @functools.partial(
        pl.kernel,
        mesh=mesh,
        out_type=jax.ShapeDtypeStruct((B * S * K, 128), jnp.float32),
        compiler_params=pltpu.CompilerParams(needs_layout_passes=False),
        scratch_types=[
            pltpu.VMEM((N,), jnp.float32),
            pltpu.VMEM((N,), jnp.float32),
            pltpu.VMEM((N,), jnp.float32),
            pltpu.VMEM((N,), jnp.float32),
            pltpu.VMEM((spw + 16,), jnp.float32),
            pltpu.VMEM((spw + 16,), jnp.float32),
            pltpu.VMEM((spw + 16,), jnp.float32),
            pltpu.VMEM((spw + 16,), jnp.float32),
            pltpu.VMEM((N + 64,), jnp.int32),
            pltpu.VMEM((128,), jnp.int32),
            pltpu.VMEM((128,), jnp.int32),
            pltpu.VMEM((256, 128), jnp.float32),
            pltpu.SemaphoreType.DMA,
        ],
    )
    def body(x_h, y_h, z_h, nx_h, ny_h, nz_h, g_h, out_h,
             xv, yv, zv, np2v, nxv, nyv, nzv, nc2v, buf, idxa, idxb, dest,
             sem):
        wid = lax.axis_index("s") * info.num_cores + lax.axis_index("c")
        b = wid // bpw
        pltpu.sync_copy(x_h.at[pl.ds(b * N, N)], xv)
        pltpu.sync_copy(y_h.at[pl.ds(b * N, N)], yv)
        pltpu.sync_copy(z_h.at[pl.ds(b * N, N)], zv)
        pltpu.sync_copy(nx_h.at[pl.ds(wid * spw, spw)], nxv.at[pl.ds(0, spw)])
        pltpu.sync_copy(ny_h.at[pl.ds(wid * spw, spw)], nyv.at[pl.ds(0, spw)])
        pltpu.sync_copy(nz_h.at[pl.ds(wid * spw, spw)], nzv.at[pl.ds(0, spw)])
        iota16 = lax.iota(jnp.int32, 16)
        grow = b * N

        def rbf16(v):
            u = lax.bitcast_convert_type(v, jnp.uint32)
            r = (u + jnp.uint32(0x7FFF) + ((u >> jnp.uint32(16))
                                           & jnp.uint32(1)))
            return lax.bitcast_convert_type(r & jnp.uint32(0xFFFF0000),
                                            jnp.float32)

        def prep_pts(c, carry):
            sl = pl.ds(c * 16, 16)
            x = xv[sl]
            y = yv[sl]
            z = zv[sl]
            np2v[sl] = x * x + y * y + z * z
            xv[sl] = rbf16(x)
            yv[sl] = rbf16(y)
            zv[sl] = rbf16(z)
            return carry

        lax.fori_loop(0, N // 16, prep_pts, jnp.int32(0))

        def prep_ctr(c, carry):
            sl = pl.ds(c * 16, 16)
            x = nxv[sl]
            y = nyv[sl]
            z = nzv[sl]
            nc2v[sl] = x * x + y * y + z * z
            nxv[sl] = rbf16(x)
            nyv[sl] = rbf16(y)
            nzv[sl] = rbf16(z)
            return carry

        lax.fori_loop(0, spw // 16, prep_ctr, jnp.int32(0))

        def per_group(g, carry):
            for c in range(8):
                s = g * 8 + c
                cx = nxv[pl.ds(s, 16)][0]
                cy = nyv[pl.ds(s, 16)][0]
                cz = nzv[pl.ds(s, 16)][0]
                nc2 = nc2v[pl.ds(s, 16)][0]
                buf[pl.ds(0, 16)] = jnp.zeros((16,), jnp.int32)

                def scan_grp(grp, cnt):
                    for u in range(4):
                        off = grp * 64 + u * 16
                        sl = pl.ds(off, 16)
                        dot = xv[sl] * cx + yv[sl] * cy + zv[sl] * cz
                        sqr = (nc2 + np2v[sl]) - 2.0 * dot
                        m = sqr <= R2
                        mi = m.astype(jnp.int32)
                        rank = jnp.cumsum(mi)
                        plsc.store_scatter(buf, [cnt + rank - 1],
                                           off + iota16, mask=m)
                        cnt = cnt + jnp.sum(mi)
                    return cnt

                cnt = lax.fori_loop(0, 16, scan_grp, jnp.int32(0))
                e2 = jnp.where(cnt < K, 32, 16)
                cnt = lax.fori_loop(16, e2, scan_grp, cnt)
                e3 = jnp.where(cnt < K, 64, 32)
                cnt = lax.fori_loop(32, e3, scan_grp, cnt)
                pv = jnp.full((16,), buf[pl.ds(0, 16)][0], jnp.int32)
                plsc.store_scatter(buf, [cnt + iota16], pv)
                plsc.store_scatter(buf, [cnt + 16 + iota16], pv)
                idx = idxa if c < 4 else idxb
                co = (c % 4) * K
                idx[pl.ds(co, 16)] = buf[pl.ds(0, 16)] + grow
                idx[pl.ds(co + 16, 16)] = buf[pl.ds(16, 16)] + grow
            cpa = pltpu.async_copy(g_h.at[idxa], dest.at[pl.ds(0, 128)], sem)
            cpb = pltpu.async_copy(g_h.at[idxb], dest.at[pl.ds(128, 128)],
                                   sem)
            cpa.wait()
            cpb.wait()
            pltpu.sync_copy(dest,
                            out_h.at[pl.ds((wid * spw + g * 8) * K, 256)])
            return carry

        lax.fori_loop(0, spw // 8, per_group, jnp.int32(0))

    return body(xf, yf, zf, nxf, nyf, nzf, gflat)


_ST = 256


def _mlp_body(g_ref, nx_ref, ny_ref, nz_ref, w1a_ref, w2_ref, b2_ref,
              w3_ref, b3_ref, out_ref):
    nxc = nx_ref[0, 0][:, None]
    nyc = ny_ref[0, 0][:, None]
    nzc = nz_ref[0, 0][:, None]
    c = (nxc * w1a_ref[pl.ds(0, 1), :]
         + nyc * w1a_ref[pl.ds(1, 1), :]
         + nzc * w1a_ref[pl.ds(2, 1), :])
    h1 = jax.nn.relu(g_ref[0][:, :, 0:C1] - c[:, None, :])
    h1f = h1.reshape(_ST * K, C1)
    h2 = jax.nn.relu(jnp.dot(h1f, w2_ref[:], preferred_element_type=jnp.float32)
                     + b2_ref[:])
    h3 = jax.nn.relu(jnp.dot(h2, w3_ref[:], preferred_element_type=jnp.float32)
                     + b3_ref[:])
    out_ref[0] = jnp.max(h3.reshape(_ST, K, 64), axis=1)


def _mlp(g4, nx, ny, nz, w1a, w2, b2r, w3, b3r):
    nt = S // _ST
    nx = nx.reshape(B * nt, 1, _ST)
    ny = ny.reshape(B * nt, 1, _ST)
    nz = nz.reshape(B * nt, 1, _ST)
    return pl.pallas_call(
        _mlp_body,
        grid=(B, nt),
        in_specs=[
            pl.BlockSpec((1, _ST, K, 128), lambda b, t: (b, t, 0, 0)),
            pl.BlockSpec((1, 1, _ST), lambda b, t: (b * nt + t, 0, 0)),
            pl.BlockSpec((1, 1, _ST), lambda b, t: (b * nt + t, 0, 0)),
            pl.BlockSpec((1, 1, _ST), lambda b, t: (b * nt + t, 0, 0)),
            pl.BlockSpec((3, C1), lambda b, t: (0, 0)),
            pl.BlockSpec((C1, 32), lambda b, t: (0, 0)),
            pl.BlockSpec((1, 32), lambda b, t: (0, 0)),
            pl.BlockSpec((32, 64), lambda b, t: (0, 0)),
            pl.BlockSpec((1, 64), lambda b, t: (0, 0)),
        ],
        out_specs=pl.BlockSpec((1, _ST, 64), lambda b, t: (b, t, 0)),
        out_shape=jax.ShapeDtypeStruct((B, S, 64), jnp.float32),
    )(g4, nx, ny, nz, w1a, w2, b2r, w3, b3r)


def kernel(xyz, points, W1, b1, W2, b2, W3, b3):
    xt = jnp.transpose(xyz, (2, 0, 1))
    newt = _fps(xt)
    new_xyz = jnp.transpose(newt, (1, 2, 0))
    w1a = W1[:3]
    w1b = W1[3:]
    g = _gtable(xyz, points, w1a, w1b, b1.reshape(1, C1))
    gathered = _sc_gather(
        xt[0].reshape(B * N), xt[1].reshape(B * N), xt[2].reshape(B * N),
        newt[0].reshape(B * S), newt[1].reshape(B * S), newt[2].reshape(B * S),
        g.reshape(B * N, 128),
    )
    g4 = gathered.reshape(B, S, K, 128)
    out = _mlp(g4, newt[0], newt[1], newt[2], w1a, W2,
               b2.reshape(1, 32), W3, b3.reshape(1, 64))
    return (new_xyz, out)


# speedup vs baseline: 19.3559x; 1.0808x over previous
import functools

import jax
import jax.numpy as jnp
from jax import lax
from jax.experimental import pallas as pl
from jax.experimental.pallas import tpu as pltpu
from jax.experimental.pallas import tpu_sc as plsc

B = 16
N = 4096
S = 1024
K = 32
R2 = 0.2 * 0.2
C1 = 32


def _emfma(y, acc):
    c = y * 4097.0
    yh = c - (c - y)
    yl = y - yh
    p = y * y
    e = ((yh * yh - p) + 2.0 * (yh * yl)) + yl * yl
    s1 = acc + p
    bb = s1 - acc
    e1 = (acc - (s1 - bb)) + (p - bb)
    return s1 + (e1 + e)


def _fps_body(xt_ref, out_ref, dist_ref):
    x = xt_ref[0]
    y = xt_ref[1]
    z = xt_ref[2]
    iota = lax.broadcasted_iota(jnp.int32, (B, N), 1)
    lane_s = lax.broadcasted_iota(jnp.int32, (B, S), 1)
    dist_ref[:] = jnp.full((B, N), 1e10, dtype=jnp.float32)

    def step(i, far):
        onehot = iota == far
        cx = jnp.sum(jnp.where(onehot, x, 0.0), axis=1, keepdims=True)
        cy = jnp.sum(jnp.where(onehot, y, 0.0), axis=1, keepdims=True)
        cz = jnp.sum(jnp.where(onehot, z, 0.0), axis=1, keepdims=True)
        dx = x - cx
        dy = y - cy
        dz = z - cz
        d = _emfma(dz, _emfma(dy, dx * dx))
        dist = jnp.minimum(dist_ref[:], d)
        dist_ref[:] = dist
        sel = lane_s == i
        cstack = jnp.stack([cx, cy, cz])
        out_ref[:] = jnp.where(sel[None], cstack, out_ref[:])
        m = jnp.max(dist, axis=1, keepdims=True)
        far = jnp.min(jnp.where(dist == m, iota, N), axis=1, keepdims=True)
        return far

    lax.fori_loop(0, S, step, jnp.zeros((B, 1), jnp.int32))


def _fps(xt):
    return pl.pallas_call(
        _fps_body,
        out_shape=jax.ShapeDtypeStruct((3, B, S), jnp.float32),
        scratch_shapes=[pltpu.VMEM((B, N), jnp.float32)],
    )(xt)


def _gtable_body(xyz_ref, pts_ref, w1a_ref, w1b_ref, b1_ref, g_ref):
    xc = xyz_ref[0, :, pl.ds(0, 1)]
    yc = xyz_ref[0, :, pl.ds(1, 1)]
    zc = xyz_ref[0, :, pl.ds(2, 1)]
    gx = (xc * w1a_ref[pl.ds(0, 1), :]
          + yc * w1a_ref[pl.ds(1, 1), :]
          + zc * w1a_ref[pl.ds(2, 1), :])
    gp = jnp.dot(pts_ref[0], w1b_ref[:], preferred_element_type=jnp.float32)
    gval = gx + gp + b1_ref[:]
    g_ref[0] = jnp.concatenate(
        [gval, jnp.zeros((N, 128 - C1), jnp.float32)], axis=1)


def _gtable(xyz, points, w1a, w1b, b1r):
    return pl.pallas_call(
        _gtable_body,
        grid=(B,),
        in_specs=[
            pl.BlockSpec((1, N, 3), lambda b: (b, 0, 0)),
            pl.BlockSpec((1, N, 32), lambda b: (b, 0, 0)),
            pl.BlockSpec((3, C1), lambda b: (0, 0)),
            pl.BlockSpec((32, C1), lambda b: (0, 0)),
            pl.BlockSpec((1, C1), lambda b: (0, 0)),
        ],
        out_specs=pl.BlockSpec((1, N, 128), lambda b: (b, 0, 0)),
        out_shape=jax.ShapeDtypeStruct((B, N, 128), jnp.float32),
    )(xyz, points, w1a, w1b, b1r)


def _sc_gather(xf, yf, zf, nxf, nyf, nzf, gflat):
    info = plsc.get_sparse_core_info()
    nw = info.num_cores * info.num_subcores
    spw = (B * S) // nw
    bpw = S // spw
    mesh = plsc.VectorSubcoreMesh(core_axis_name="c", subcore_axis_name="s")

    @functools.partial(
        pl.kernel,
        mesh=mesh,
        out_type=jax.ShapeDtypeStruct((B * S * K, 128), jnp.float32),
        compiler_params=pltpu.CompilerParams(needs_layout_passes=False),
        scratch_types=[
            pltpu.VMEM((N,), jnp.float32),
            pltpu.VMEM((N,), jnp.float32),
            pltpu.VMEM((N,), jnp.float32),
            pltpu.VMEM((N,), jnp.float32),
            pltpu.VMEM((spw + 16,), jnp.float32),
            pltpu.VMEM((spw + 16,), jnp.float32),
            pltpu.VMEM((spw + 16,), jnp.float32),
            pltpu.VMEM((spw + 16,), jnp.float32),
            pltpu.VMEM((N + 64,), jnp.int32),
            pltpu.VMEM((2, 128), jnp.int32),
            pltpu.VMEM((2, 128, 128), jnp.float32),
            pltpu.SemaphoreType.DMA,
            pltpu.SemaphoreType.DMA,
        ],
    )
    def body(x_h, y_h, z_h, nx_h, ny_h, nz_h, g_h, out_h,
             xv, yv, zv, np2v, nxv, nyv, nzv, nc2v, buf, idxq, dest,
             sema, semb):
        wid = lax.axis_index("s") * info.num_cores + lax.axis_index("c")
        b = wid // bpw
        pltpu.sync_copy(x_h.at[pl.ds(b * N, N)], xv)
        pltpu.sync_copy(y_h.at[pl.ds(b * N, N)], yv)
        pltpu.sync_copy(z_h.at[pl.ds(b * N, N)], zv)
        pltpu.sync_copy(nx_h.at[pl.ds(wid * spw, spw)], nxv.at[pl.ds(0, spw)])
        pltpu.sync_copy(ny_h.at[pl.ds(wid * spw, spw)], nyv.at[pl.ds(0, spw)])
        pltpu.sync_copy(nz_h.at[pl.ds(wid * spw, spw)], nzv.at[pl.ds(0, spw)])
        iota16 = lax.iota(jnp.int32, 16)
        grow = b * N

        def rbf16(v):
            u = lax.bitcast_convert_type(v, jnp.uint32)
            r = (u + jnp.uint32(0x7FFF) + ((u >> jnp.uint32(16))
                                           & jnp.uint32(1)))
            return lax.bitcast_convert_type(r & jnp.uint32(0xFFFF0000),
                                            jnp.float32)

        def prep_pts(c, carry):
            sl = pl.ds(c * 16, 16)
            x = xv[sl]
            y = yv[sl]
            z = zv[sl]
            np2v[sl] = x * x + y * y + z * z
            xv[sl] = rbf16(x)
            yv[sl] = rbf16(y)
            zv[sl] = rbf16(z)
            return carry

        lax.fori_loop(0, N // 16, prep_pts, jnp.int32(0))

        def prep_ctr(c, carry):
            sl = pl.ds(c * 16, 16)
            x = nxv[sl]
            y = nyv[sl]
            z = nzv[sl]
            nc2v[sl] = x * x + y * y + z * z
            nxv[sl] = rbf16(x)
            nyv[sl] = rbf16(y)
            nzv[sl] = rbf16(z)
            return carry

        lax.fori_loop(0, spw // 16, prep_ctr, jnp.int32(0))

        def scan_sel(q, p):
            for c in range(4):
                s = q * 4 + c
                cx = nxv[pl.ds(s, 16)][0]
                cy = nyv[pl.ds(s, 16)][0]
                cz = nzv[pl.ds(s, 16)][0]
                nc2 = nc2v[pl.ds(s, 16)][0]
                buf[pl.ds(0, 16)] = jnp.zeros((16,), jnp.int32)

                def scan_grp(grp, cnt):
                    for u in range(2):
                        off = grp * 32 + u * 16
                        sl = pl.ds(off, 16)
                        dot = xv[sl] * cx + yv[sl] * cy + zv[sl] * cz
                        sqr = (nc2 + np2v[sl]) - 2.0 * dot
                        m = sqr <= R2
                        mi = m.astype(jnp.int32)
                        rank = jnp.cumsum(mi)
                        plsc.store_scatter(buf, [cnt + rank - 1],
                                           off + iota16, mask=m)
                        cnt = cnt + rank[15]
                    return cnt

                cnt = lax.fori_loop(0, 32, scan_grp, jnp.int32(0))
                e2 = jnp.where(cnt < K, 64, 32)
                cnt = lax.fori_loop(32, e2, scan_grp, cnt)
                e3 = jnp.where(cnt < K, 128, 64)
                cnt = lax.fori_loop(64, e3, scan_grp, cnt)
                pv = jnp.full((16,), buf[pl.ds(0, 16)][0], jnp.int32)
                plsc.store_scatter(buf, [cnt + iota16], pv)
                plsc.store_scatter(buf, [cnt + 16 + iota16], pv)
                co = c * K
                idxq.at[p][pl.ds(co, 16)] = buf[pl.ds(0, 16)] + grow
                idxq.at[p][pl.ds(co + 16, 16)] = buf[pl.ds(16, 16)] + grow

        def fire(p, sem):
            pltpu.async_copy(g_h.at[idxq.at[p]], dest.at[p], sem)

        def drain(q, p, sem):
            pltpu.make_async_copy(g_h.at[idxq.at[p]], dest.at[p], sem).wait()
            pltpu.sync_copy(dest.at[p],
                            out_h.at[pl.ds((wid * spw + q * 4) * K, 128)])

        nq = spw // 4
        scan_sel(jnp.int32(0), 0)
        fire(0, sema)

        def pair(h, carry):
            q1 = 2 * h + 1
            scan_sel(q1, 1)
            fire(1, semb)
            drain(q1 - 1, 0, sema)
            q2 = 2 * h + 2
            scan_sel(q2, 0)
            fire(0, sema)
            drain(q2 - 1, 1, semb)
            return carry

        lax.fori_loop(0, (nq - 2) // 2, pair, jnp.int32(0))
        scan_sel(jnp.int32(nq - 1), 1)
        fire(1, semb)
        drain(jnp.int32(nq - 2), 0, sema)
        drain(jnp.int32(nq - 1), 1, semb)

    return body(xf, yf, zf, nxf, nyf, nzf, gflat)


_ST = 256


def _mlp_body(g_ref, nx_ref, ny_ref, nz_ref, w1a_ref, w2_ref, b2_ref,
              w3_ref, b3_ref, out_ref):
    nxc = nx_ref[0, 0][:, None]
    nyc = ny_ref[0, 0][:, None]
    nzc = nz_ref[0, 0][:, None]
    c = (nxc * w1a_ref[pl.ds(0, 1), :]
         + nyc * w1a_ref[pl.ds(1, 1), :]
         + nzc * w1a_ref[pl.ds(2, 1), :])
    h1 = jax.nn.relu(g_ref[0][:, :, 0:C1] - c[:, None, :])
    h1f = h1.reshape(_ST * K, C1)
    h2 = jax.nn.relu(jnp.dot(h1f, w2_ref[:], preferred_element_type=jnp.float32)
                     + b2_ref[:])
    h3 = jax.nn.relu(jnp.dot(h2, w3_ref[:], preferred_element_type=jnp.float32)
                     + b3_ref[:])
    out_ref[0] = jnp.max(h3.reshape(_ST, K, 64), axis=1)


def _mlp(g4, nx, ny, nz, w1a, w2, b2r, w3, b3r):
    nt = S // _ST
    nx = nx.reshape(B * nt, 1, _ST)
    ny = ny.reshape(B * nt, 1, _ST)
    nz = nz.reshape(B * nt, 1, _ST)
    return pl.pallas_call(
        _mlp_body,
        grid=(B, nt),
        in_specs=[
            pl.BlockSpec((1, _ST, K, 128), lambda b, t: (b, t, 0, 0)),
            pl.BlockSpec((1, 1, _ST), lambda b, t: (b * nt + t, 0, 0)),
            pl.BlockSpec((1, 1, _ST), lambda b, t: (b * nt + t, 0, 0)),
            pl.BlockSpec((1, 1, _ST), lambda b, t: (b * nt + t, 0, 0)),
            pl.BlockSpec((3, C1), lambda b, t: (0, 0)),
            pl.BlockSpec((C1, 32), lambda b, t: (0, 0)),
            pl.BlockSpec((1, 32), lambda b, t: (0, 0)),
            pl.BlockSpec((32, 64), lambda b, t: (0, 0)),
            pl.BlockSpec((1, 64), lambda b, t: (0, 0)),
        ],
        out_specs=pl.BlockSpec((1, _ST, 64), lambda b, t: (b, t, 0)),
        out_shape=jax.ShapeDtypeStruct((B, S, 64), jnp.float32),
    )(g4, nx, ny, nz, w1a, w2, b2r, w3, b3r)


def kernel(xyz, points, W1, b1, W2, b2, W3, b3):
    xt = jnp.transpose(xyz, (2, 0, 1))
    newt = _fps(xt)
    new_xyz = jnp.transpose(newt, (1, 2, 0))
    w1a = W1[:3]
    w1b = W1[3:]
    g = _gtable(xyz, points, w1a, w1b, b1.reshape(1, C1))
    gathered = _sc_gather(
        xt[0].reshape(B * N), xt[1].reshape(B * N), xt[2].reshape(B * N),
        newt[0].reshape(B * S), newt[1].reshape(B * S), newt[2].reshape(B * S),
        g.reshape(B * N, 128),
    )
    g4 = gathered.reshape(B, S, K, 128)
    out = _mlp(g4, newt[0], newt[1], newt[2], w1a, W2,
               b2.reshape(1, 32), W3, b3.reshape(1, 64))
    return (new_xyz, out)
